# Initial kernel scaffold; baseline (speedup 1.0000x reference)
import jax
import jax.numpy as jnp
from jax.experimental import pallas as pl


def kernel(input_ids, role_ids, token_table, role_table, input_bias, ln_gamma, ln_beta):
    raise NotImplementedError("write your pallas kernel here")


# speedup vs baseline: 1.5482x; 1.5482x over previous
import functools

import jax
import jax.numpy as jnp
from jax import lax
from jax.experimental import pallas as pl
from jax.experimental.pallas import tpu as pltpu
from jax.experimental.pallas import tpu_sc as plsc

_D = 768
_B = 32768
_EPS = 1e-5

_NC = 2
_NS = 16
_NW = _NC * _NS
_BPW = _B // _NW
_CHUNK = 64
_NCHUNK = _BPW // _CHUNK


def _sc_gather_body(table_hbm, ids_hbm, out_hbm, idx_v, rows_v, sem):
    wid = lax.axis_index("s") * _NC + lax.axis_index("c")
    pltpu.sync_copy(ids_hbm.at[wid], idx_v)
    base = wid * _BPW
    for j in range(_NCHUNK):
        pltpu.async_copy(table_hbm.at[idx_v.at[j]], rows_v, sem).wait()
        pltpu.sync_copy(rows_v, out_hbm.at[pl.ds(base + j * _CHUNK, _CHUNK)])


_sc_gather = functools.partial(
    pl.kernel,
    out_type=jax.ShapeDtypeStruct((_B, _D), jnp.float32),
    mesh=plsc.VectorSubcoreMesh(core_axis_name="c", subcore_axis_name="s"),
    scratch_types=[
        pltpu.VMEM((_NCHUNK, _CHUNK), jnp.int32),
        pltpu.VMEM((_CHUNK, _D), jnp.float32),
        pltpu.SemaphoreType.DMA,
    ],
)(_sc_gather_body)


_RBLK = 256


def _tc_ln_body(rows_ref, rid_ref, role_ref, bias_ref, gamma_ref, beta_ref, out_ref):
    y = rows_ref[...]
    rid = rid_ref[...]
    rb = role_ref[...] + bias_ref[...]
    contrib = jnp.broadcast_to(rb[0:1, :], y.shape)
    for k in range(1, 4):
        contrib = jnp.where(rid == k, rb[k:k + 1, :], contrib)
    y = y + contrib
    mean = jnp.mean(y, axis=1, keepdims=True)
    yc = y - mean
    var = jnp.mean(yc * yc, axis=1, keepdims=True)
    normed = yc * lax.rsqrt(var + _EPS)
    out_ref[...] = normed * gamma_ref[...] + beta_ref[...]


def _tc_ln(rows, rid2d, role_table, bias2d, gamma2d, beta2d):
    grid = _B // _RBLK
    return pl.pallas_call(
        _tc_ln_body,
        grid=(grid,),
        in_specs=[
            pl.BlockSpec((_RBLK, _D), lambda i: (i, 0)),
            pl.BlockSpec((_RBLK, 1), lambda i: (i, 0)),
            pl.BlockSpec((4, _D), lambda i: (0, 0)),
            pl.BlockSpec((1, _D), lambda i: (0, 0)),
            pl.BlockSpec((1, _D), lambda i: (0, 0)),
            pl.BlockSpec((1, _D), lambda i: (0, 0)),
        ],
        out_specs=pl.BlockSpec((_RBLK, _D), lambda i: (i, 0)),
        out_shape=jax.ShapeDtypeStruct((_B, _D), jnp.float32),
    )(rows, rid2d, role_table, bias2d, gamma2d, beta2d)


def kernel(input_ids, role_ids, token_table, role_table, input_bias, ln_gamma, ln_beta):
    ids = input_ids.reshape(_NW, _NCHUNK, _CHUNK).astype(jnp.int32)
    gathered = _sc_gather(token_table, ids)
    rid2d = role_ids.reshape(_B, 1).astype(jnp.int32)
    out = _tc_ln(
        gathered,
        rid2d,
        role_table,
        input_bias.reshape(1, _D),
        ln_gamma.reshape(1, _D),
        ln_beta.reshape(1, _D),
    )
    return out.reshape(input_ids.shape[0], input_ids.shape[1], _D)


# speedup vs baseline: 1.9028x; 1.2291x over previous
import functools

import jax
import jax.numpy as jnp
from jax import lax
from jax.experimental import pallas as pl
from jax.experimental.pallas import tpu as pltpu
from jax.experimental.pallas import tpu_sc as plsc

_D = 768
_B = 32768
_EPS = 1e-5

_NC = 2
_NS = 16
_NW = _NC * _NS
_BPW = _B // _NW
_CHUNK = 64
_NCHUNK = _BPW // _CHUNK


def _sc_gather_body(table_hbm, ids_hbm, out_hbm, idx_v, rows0, rows1, sem0, sem1):
    wid = lax.axis_index("s") * _NC + lax.axis_index("c")
    pltpu.sync_copy(ids_hbm.at[wid], idx_v)
    base = wid * _BPW
    bufs = (rows0, rows1)
    sems = (sem0, sem1)
    pltpu.async_copy(table_hbm.at[idx_v.at[0]], bufs[0], sems[0])
    for j in range(_NCHUNK):
        cur = j % 2
        if j + 1 < _NCHUNK:
            pltpu.async_copy(table_hbm.at[idx_v.at[j + 1]], bufs[1 - cur], sems[1 - cur])
        pltpu.make_async_copy(table_hbm.at[idx_v.at[j]], bufs[cur], sems[cur]).wait()
        pltpu.sync_copy(bufs[cur], out_hbm.at[pl.ds(base + j * _CHUNK, _CHUNK)])


_sc_gather = functools.partial(
    pl.kernel,
    out_type=jax.ShapeDtypeStruct((_B, _D), jnp.float32),
    mesh=plsc.VectorSubcoreMesh(core_axis_name="c", subcore_axis_name="s"),
    scratch_types=[
        pltpu.VMEM((_NCHUNK, _CHUNK), jnp.int32),
        pltpu.VMEM((_CHUNK, _D), jnp.float32),
        pltpu.VMEM((_CHUNK, _D), jnp.float32),
        pltpu.SemaphoreType.DMA,
        pltpu.SemaphoreType.DMA,
    ],
)(_sc_gather_body)


_RBLK = 512


def _tc_ln_body(rows_ref, rid_ref, role_ref, bias_ref, gamma_ref, beta_ref, out_ref):
    y = rows_ref[...]
    rid = rid_ref[...]
    rb = role_ref[...] + bias_ref[...]
    contrib = jnp.broadcast_to(rb[0:1, :], y.shape)
    for k in range(1, 4):
        contrib = jnp.where(rid == k, rb[k:k + 1, :], contrib)
    y = y + contrib
    mean = jnp.mean(y, axis=1, keepdims=True)
    yc = y - mean
    var = jnp.mean(yc * yc, axis=1, keepdims=True)
    normed = yc * lax.rsqrt(var + _EPS)
    out_ref[...] = normed * gamma_ref[...] + beta_ref[...]


def _tc_ln(rows, rid2d, role_table, bias2d, gamma2d, beta2d):
    grid = _B // _RBLK
    return pl.pallas_call(
        _tc_ln_body,
        grid=(grid,),
        in_specs=[
            pl.BlockSpec((_RBLK, _D), lambda i: (i, 0)),
            pl.BlockSpec((_RBLK, 1), lambda i: (i, 0)),
            pl.BlockSpec((4, _D), lambda i: (0, 0)),
            pl.BlockSpec((1, _D), lambda i: (0, 0)),
            pl.BlockSpec((1, _D), lambda i: (0, 0)),
            pl.BlockSpec((1, _D), lambda i: (0, 0)),
        ],
        out_specs=pl.BlockSpec((_RBLK, _D), lambda i: (i, 0)),
        out_shape=jax.ShapeDtypeStruct((_B, _D), jnp.float32),
    )(rows, rid2d, role_table, bias2d, gamma2d, beta2d)


def kernel(input_ids, role_ids, token_table, role_table, input_bias, ln_gamma, ln_beta):
    ids = input_ids.reshape(_NW, _NCHUNK, _CHUNK).astype(jnp.int32)
    gathered = _sc_gather(token_table, ids)
    rid2d = role_ids.reshape(_B, 1).astype(jnp.int32)
    out = _tc_ln(
        gathered,
        rid2d,
        role_table,
        input_bias.reshape(1, _D),
        ln_gamma.reshape(1, _D),
        ln_beta.reshape(1, _D),
    )
    return out.reshape(input_ids.shape[0], input_ids.shape[1], _D)
